# baseline (speedup 1.0000x reference)
import jax
import jax.numpy as jnp
from jax import lax
from jax.experimental import pallas as pl


def _pair_pool_matrix(n):
    r = lax.broadcasted_iota(jnp.int32, (n, n // 2), 0)
    c = lax.broadcasted_iota(jnp.int32, (n, n // 2), 1)
    return jnp.where(r // 2 == c, jnp.float32(0.5), jnp.float32(0.0))


def _pool2(x):
    C, H, W = x.shape
    m_col = _pair_pool_matrix(W)
    p_row = _pair_pool_matrix(H).T
    outs = []
    for ch in range(C):
        t = jnp.dot(p_row, x[ch], preferred_element_type=jnp.float32)
        outs.append(jnp.dot(t, m_col, preferred_element_type=jnp.float32))
    return jnp.stack(outs, axis=0)


def _full_rows(canvas, r0, n):
    s = canvas[:, 16 * r0:16 * (r0 + n), :]
    return s.reshape(3, n, 16, 128).transpose(1, 0, 2, 3)


def _mid_rows(canvas):
    rows = []
    for r in (2, 4):
        a = canvas[:, 16 * r:16 * (r + 1), :]
        b = canvas[:, 16 * (r + 1):16 * (r + 2), :]
        rows.append(jnp.concatenate(
            [a[:, :, 0:32], a[:, :, 96:128],
             b[:, :, 0:32], b[:, :, 96:128]], axis=2))
    return jnp.stack(rows, axis=0)


def _ring_rows(canvas):
    return jnp.concatenate([
        _full_rows(canvas, 0, 2),
        _mid_rows(canvas),
        _full_rows(canvas, 6, 2),
    ], axis=0)


def _body(x_ref, o_ref):
    x = x_ref[0]
    f2 = _pool2(x)
    f4 = _pool2(f2)
    lvl0 = _full_rows(x[:, 192:320, 192:320], 0, 8)
    lvl1 = _ring_rows(f2[:, 64:192, 64:192])
    lvl2 = _ring_rows(f4)
    o_ref[0] = jnp.concatenate([lvl0, lvl1, lvl2], axis=0)


def kernel(images):
    B, C, H, W = images.shape
    packed = pl.pallas_call(
        _body,
        grid=(B,),
        in_specs=[pl.BlockSpec((1, C, H, W), lambda b: (b, 0, 0, 0))],
        out_specs=pl.BlockSpec((1, 20, 3, 16, 128),
                               lambda b: (b, 0, 0, 0, 0)),
        out_shape=jax.ShapeDtypeStruct((B, 20, 3, 16, 128), jnp.float32),
    )(images)
    out = packed.reshape(B, 20, 3, 16, 8, 16)
    out = out.transpose(0, 1, 4, 2, 3, 5)
    return out.reshape(B, 160, 3, 16, 16)


# speedup vs baseline: 1.0611x; 1.0611x over previous
import jax
import jax.numpy as jnp
from jax import lax
from jax.experimental import pallas as pl


def _pair_pool_matrix(n):
    r = lax.broadcasted_iota(jnp.int32, (n, n // 2), 0)
    c = lax.broadcasted_iota(jnp.int32, (n, n // 2), 1)
    return jnp.where(r // 2 == c, jnp.float32(0.5), jnp.float32(0.0))


def _pool2(x):
    C, H, W = x.shape
    m_col = _pair_pool_matrix(W)
    p_row = _pair_pool_matrix(H).T
    outs = []
    for ch in range(C):
        t = jnp.dot(p_row, x[ch], preferred_element_type=jnp.float32)
        outs.append(jnp.dot(t, m_col, preferred_element_type=jnp.float32))
    return jnp.stack(outs, axis=0)


def _grid_tokens(canvas):
    cols = [canvas[:, :, 16 * gc:16 * (gc + 1)] for gc in range(8)]
    g = jnp.stack(cols, axis=0)
    g = g.reshape(8, 3, 8, 16, 16).transpose(2, 0, 1, 3, 4)
    return g.reshape(64, 3, 16, 16)


def _ring_tokens(canvas):
    g = _grid_tokens(canvas)
    mids = [g[i:i + 2] for i in (16, 22, 24, 30, 32, 38, 40, 46)]
    return jnp.concatenate([g[0:16]] + mids + [g[48:64]], axis=0)


def _body(x_ref, o_ref):
    x = x_ref[0]
    f2 = _pool2(x)
    f4 = _pool2(f2)
    lvl0 = _grid_tokens(x[:, 192:320, 192:320])
    lvl1 = _ring_tokens(f2[:, 64:192, 64:192])
    lvl2 = _ring_tokens(f4)
    o_ref[0] = jnp.concatenate([lvl0, lvl1, lvl2], axis=0)


def kernel(images):
    B, C, H, W = images.shape
    return pl.pallas_call(
        _body,
        grid=(B,),
        in_specs=[pl.BlockSpec((1, C, H, W), lambda b: (b, 0, 0, 0))],
        out_specs=pl.BlockSpec((1, 160, 3, 16, 16),
                               lambda b: (b, 0, 0, 0, 0)),
        out_shape=jax.ShapeDtypeStruct((B, 160, 3, 16, 16), jnp.float32),
    )(images)
